# baseline (speedup 1.0000x reference)
import functools

import jax
import jax.numpy as jnp
from jax import lax
from jax.experimental import pallas as pl
from jax.experimental.pallas import tpu as pltpu
from jax.experimental.pallas import tpu_sc as plsc

N = 10000
E = 320000
D = 128
H = 128
ED = 16
G = 64

NC = 2
NS = 16
NW = NC * NS
EPW = E // NW
C = 40
NCHUNK = EPW // C
RPS = 624
RTAIL = N - NS * RPS


def _make_edge_gather_scatter(with_norm: bool):
    mesh = plsc.VectorSubcoreMesh(core_axis_name="c", subcore_axis_name="s")

    scratch = [
        pltpu.VMEM((EPW,), jnp.int32),
        [pltpu.VMEM((C,), jnp.int32) for _ in range(3)],
        [pltpu.VMEM((C, D), jnp.float32) for _ in range(3)],
        [pltpu.VMEM((C, D), jnp.float32) for _ in range(3)],
        pltpu.VMEM((8, D), jnp.float32),
        pltpu.VMEM_SHARED((N, D), jnp.float32),
        pltpu.SemaphoreType.DMA((5, 3)),
    ]

    @functools.partial(
        pl.kernel,
        out_type=jax.ShapeDtypeStruct((NC, N, D), jnp.float32),
        mesh=mesh,
        scratch_types=scratch,
    )
    def k(h_hbm, sgb_hbm, e_hbm, src_hbm, dst_hbm, z_hbm, out_hbm,
          srcall_v, dst_v, hrow_v, e_v, sgb_v, aggr_sh, sems):
        c = lax.axis_index("c")
        s = lax.axis_index("s")
        wid = s * NC + c
        base = wid * EPW
        pltpu.sync_copy(src_hbm.at[pl.ds(base, EPW)], srcall_v)
        pltpu.sync_copy(sgb_hbm, sgb_v)
        pltpu.sync_copy(z_hbm.at[pl.ds(s * RPS, RPS)],
                        aggr_sh.at[pl.ds(s * RPS, RPS)])

        @pl.when(s == NS - 1)
        def _():
            pltpu.sync_copy(z_hbm.at[pl.ds(NS * RPS, RTAIL)],
                            aggr_sh.at[pl.ds(NS * RPS, RTAIL)])

        plsc.subcore_barrier()

        def gather_desc(j, b):
            return pltpu.make_async_copy(
                h_hbm.at[srcall_v.at[pl.ds(j * C, C)]], hrow_v[b],
                sems.at[0, b])

        def e_desc(j, b):
            return pltpu.make_async_copy(
                e_hbm.at[pl.ds(base + j * C, C)], e_v[b],
                sems.at[1, b])

        def dst_desc(j, b):
            return pltpu.make_async_copy(
                dst_hbm.at[pl.ds(base + j * C, C)], dst_v[b],
                sems.at[2, b])

        def scat_wait(b):
            pltpu.make_async_copy(
                hrow_v[b], aggr_sh.at[dst_v[b]], sems.at[3, b]).wait()

        def issue(j, b):
            dst_desc(j, b).start()
            e_desc(j, b).start()
            gather_desc(j, b).start()

        def consume(j, b):
            gather_desc(j, b).wait()
            e_desc(j, b).wait()

            @plsc.parallel_loop(0, C, unroll=4)
            def _(i):
                for col in range(D // 16):
                    sl = pl.ds(col * 16, 16)
                    hv = hrow_v[b][i, sl]
                    if with_norm:
                        hv = jnp.maximum(hv * sgb_v[0, sl] + sgb_v[1, sl],
                                         0.0)
                    hrow_v[b][i, sl] = jnp.maximum(hv + e_v[b][i, sl], 0.0)

            dst_desc(j, b).wait()
            pltpu.async_copy(hrow_v[b], aggr_sh.at[dst_v[b]],
                             sems.at[3, b], add=True)

        issue(0, 0)
        issue(1, 1)

        def trip(i, carry):
            j0 = 3 * i
            for t in range(3):
                j = j0 + t
                consume(j, t)
                b2 = (t + 2) % 3

                @pl.when((j >= 1) & (j + 2 < NCHUNK))
                def _():
                    scat_wait(b2)

                @pl.when(j + 2 < NCHUNK)
                def _():
                    issue(j + 2, b2)

            return carry

        lax.fori_loop(0, NCHUNK // 3, trip, 0)
        consume(NCHUNK - 1, (NCHUNK - 1) % 3)
        scat_wait((NCHUNK - 3) % 3)
        scat_wait((NCHUNK - 2) % 3)
        scat_wait((NCHUNK - 1) % 3)
        plsc.subcore_barrier()
        pltpu.sync_copy(aggr_sh.at[pl.ds(s * RPS, RPS)],
                        out_hbm.at[c, pl.ds(s * RPS, RPS)])

        @pl.when(s == NS - 1)
        def _():
            pltpu.sync_copy(aggr_sh.at[pl.ds(NS * RPS, RTAIL)],
                            out_hbm.at[c, pl.ds(NS * RPS, RTAIL)])

    return k


_EDGE_GS_PLAIN = _make_edge_gather_scatter(False)
_EDGE_GS_NORM = _make_edge_gather_scatter(True)


BE = 4000


def _edge_lin_body(attr_ref, we_ref, be_ref, out_ref):
    out_ref[...] = (jnp.dot(attr_ref[...], we_ref[...],
                            preferred_element_type=jnp.float32)
                    + be_ref[...])


def _edge_lin(edge_attr, we, be):
    return pl.pallas_call(
        _edge_lin_body,
        grid=(E // BE,),
        in_specs=[
            pl.BlockSpec((BE, ED), lambda i: (i, 0)),
            pl.BlockSpec((ED, D), lambda i: (0, 0)),
            pl.BlockSpec((1, D), lambda i: (0, 0)),
        ],
        out_specs=pl.BlockSpec((BE, D), lambda i: (i, 0)),
        out_shape=jax.ShapeDtypeStruct((E, D), jnp.float32),
    )(edge_attr, we, be.reshape(1, D))


BN = 1000
NB = N // BN


def _make_node_mlp(with_norm: bool):
    def body(h_ref, sgb_ref, p_ref, wa_ref, ba_ref, wb_ref, bb_ref,
             g_ref, bt_ref, y_ref, so_ref):
        i = pl.program_id(0)
        hv = h_ref[...]
        if with_norm:
            hv = jnp.maximum(hv * sgb_ref[0:1, :] + sgb_ref[1:2, :], 0.0)
        hpre = hv + p_ref[0] + p_ref[1]
        t = jnp.maximum(
            jnp.dot(hpre, wa_ref[...], preferred_element_type=jnp.float32)
            + ba_ref[...], 0.0)
        y = (jnp.dot(t, wb_ref[...], preferred_element_type=jnp.float32)
             + bb_ref[...])
        y_ref[...] = y
        s1 = jnp.sum(y, axis=0, keepdims=True)
        s2 = jnp.sum(y * y, axis=0, keepdims=True)
        upd = jnp.concatenate([s1, s2, jnp.zeros((6, H), jnp.float32)],
                              axis=0)

        @pl.when(i == 0)
        def _():
            so_ref[...] = upd

        @pl.when(i > 0)
        def _():
            so_ref[...] = so_ref[...] + upd

        @pl.when(i == NB - 1)
        def _():
            mu = so_ref[0:1, :] * (1.0 / N)
            var = so_ref[1:2, :] * (1.0 / N) - mu * mu
            scale = lax.rsqrt(var + 1e-5) * g_ref[...]
            bias = bt_ref[...] - mu * scale
            so_ref[...] = jnp.concatenate(
                [scale, bias, jnp.zeros((6, H), jnp.float32)], axis=0)

    def call(h, sgb, p, wa, ba, wb, bb, g, bt):
        return pl.pallas_call(
            body,
            grid=(NB,),
            in_specs=[
                pl.BlockSpec((BN, D), lambda i: (i, 0)),
                pl.BlockSpec((8, D), lambda i: (0, 0)),
                pl.BlockSpec((NC, BN, D), lambda i: (0, i, 0)),
                pl.BlockSpec((D, H), lambda i: (0, 0)),
                pl.BlockSpec((1, H), lambda i: (0, 0)),
                pl.BlockSpec((H, H), lambda i: (0, 0)),
                pl.BlockSpec((1, H), lambda i: (0, 0)),
                pl.BlockSpec((1, H), lambda i: (0, 0)),
                pl.BlockSpec((1, H), lambda i: (0, 0)),
            ],
            out_specs=[
                pl.BlockSpec((BN, H), lambda i: (i, 0)),
                pl.BlockSpec((8, H), lambda i: (0, 0)),
            ],
            out_shape=[
                jax.ShapeDtypeStruct((N, H), jnp.float32),
                jax.ShapeDtypeStruct((8, H), jnp.float32),
            ],
        )(h, sgb, p, wa, ba.reshape(1, H), wb, bb.reshape(1, H),
          g.reshape(1, H), bt.reshape(1, H))

    return call


_NODE_MLP_PLAIN = _make_node_mlp(False)
_NODE_MLP_NORM = _make_node_mlp(True)


def _pool_body(y_ref, sgb_ref, b_ref, wo_ref, bo_ref, out_ref,
               acc_ref, cnt_ref):
    i = pl.program_id(0)
    h = jnp.maximum(y_ref[...] * sgb_ref[0:1, :] + sgb_ref[1:2, :], 0.0)
    b = b_ref[0, 0, :]
    gids = lax.broadcasted_iota(jnp.int32, (G, BN), 0)
    onehot = (b[None, :] == gids).astype(jnp.float32)
    pacc = jnp.dot(onehot, h, preferred_element_type=jnp.float32)
    pcnt = jnp.sum(onehot, axis=1, keepdims=True)

    @pl.when(i == 0)
    def _():
        acc_ref[...] = pacc
        cnt_ref[...] = pcnt

    @pl.when(i > 0)
    def _():
        acc_ref[...] = acc_ref[...] + pacc
        cnt_ref[...] = cnt_ref[...] + pcnt

    @pl.when(i == NB - 1)
    def _():
        pooled = acc_ref[...] / jnp.maximum(cnt_ref[...], 1.0)
        out_ref[...] = (jnp.dot(pooled, wo_ref[...],
                                preferred_element_type=jnp.float32)
                        + bo_ref[...])


def _pool(y, sgb, batch, wo, bo):
    b3 = batch.reshape(NB, 1, BN)
    return pl.pallas_call(
        _pool_body,
        grid=(NB,),
        in_specs=[
            pl.BlockSpec((BN, H), lambda i: (i, 0)),
            pl.BlockSpec((8, H), lambda i: (0, 0)),
            pl.BlockSpec((1, 1, BN), lambda i: (i, 0, 0)),
            pl.BlockSpec((H, 1), lambda i: (0, 0)),
            pl.BlockSpec((1, 1), lambda i: (0, 0)),
        ],
        out_specs=pl.BlockSpec((G, 1), lambda i: (0, 0)),
        out_shape=jax.ShapeDtypeStruct((G, 1), jnp.float32),
        scratch_shapes=[
            pltpu.VMEM((G, H), jnp.float32),
            pltpu.VMEM((G, 1), jnp.float32),
        ],
    )(y, sgb, b3, wo, bo.reshape(1, 1))


def kernel(x, edge_index, edge_attr, batch,
           We1, be1, W1a, b1a, W1b, b1b, gamma1, beta1,
           We2, be2, W2a, b2a, W2b, b2b, gamma2, beta2,
           We3, be3, W3a, b3a, W3b, b3b, gamma3, beta3,
           Wout, bout):
    src = edge_index[0]
    dst = edge_index[1]
    zeros = jnp.zeros((N, D), jnp.float32)
    sgb0 = jnp.zeros((8, D), jnp.float32)

    e1 = _edge_lin(edge_attr, We1, be1)
    p1 = _EDGE_GS_PLAIN(x, sgb0, e1, src, dst, zeros)
    e2 = _edge_lin(edge_attr, We2, be2)
    y1, sgb1 = _NODE_MLP_PLAIN(x, sgb0, p1, W1a, b1a, W1b, b1b,
                               gamma1, beta1)
    p2 = _EDGE_GS_NORM(y1, sgb1, e2, src, dst, zeros)
    e3 = _edge_lin(edge_attr, We3, be3)
    y2, sgb2 = _NODE_MLP_NORM(y1, sgb1, p2, W2a, b2a, W2b, b2b,
                              gamma2, beta2)
    p3 = _EDGE_GS_NORM(y2, sgb2, e3, src, dst, zeros)
    y3, sgb3 = _NODE_MLP_NORM(y2, sgb2, p3, W3a, b3a, W3b, b3b,
                              gamma3, beta3)
    return _pool(y3, sgb3, batch, Wout, bout)


# speedup vs baseline: 1.0591x; 1.0591x over previous
import functools

import jax
import jax.numpy as jnp
from jax import lax
from jax.experimental import pallas as pl
from jax.experimental.pallas import tpu as pltpu
from jax.experimental.pallas import tpu_sc as plsc

N = 10000
E = 320000
D = 128
H = 128
ED = 16
G = 64

NC = 2
NS = 16
NW = NC * NS
EPW = E // NW
C = 40
NCHUNK = EPW // C
RPS = 624
RTAIL = N - NS * RPS


def _make_edge_gather_scatter(with_norm: bool):
    mesh = plsc.VectorSubcoreMesh(core_axis_name="c", subcore_axis_name="s")

    scratch = [
        pltpu.VMEM((EPW,), jnp.int32),
        [pltpu.VMEM((C,), jnp.int32) for _ in range(3)],
        [pltpu.VMEM((C, D), jnp.float32) for _ in range(3)],
        [pltpu.VMEM((C, D), jnp.float32) for _ in range(3)],
        pltpu.VMEM((8, D), jnp.float32),
        pltpu.VMEM_SHARED((N, D), jnp.float32),
        pltpu.SemaphoreType.DMA((5, 3)),
    ]

    @functools.partial(
        pl.kernel,
        out_type=jax.ShapeDtypeStruct((NC, N, D), jnp.float32),
        mesh=mesh,
        scratch_types=scratch,
    )
    def k(h_hbm, sgb_hbm, e_hbm, src_hbm, dst_hbm, z_hbm, out_hbm,
          srcall_v, dst_v, hrow_v, e_v, sgb_v, aggr_sh, sems):
        c = lax.axis_index("c")
        s = lax.axis_index("s")
        wid = s * NC + c
        base = wid * EPW
        pltpu.sync_copy(src_hbm.at[pl.ds(base, EPW)], srcall_v)
        pltpu.sync_copy(sgb_hbm, sgb_v)
        pltpu.sync_copy(z_hbm.at[pl.ds(s * RPS, RPS)],
                        aggr_sh.at[pl.ds(s * RPS, RPS)])

        @pl.when(s == NS - 1)
        def _():
            pltpu.sync_copy(z_hbm.at[pl.ds(NS * RPS, RTAIL)],
                            aggr_sh.at[pl.ds(NS * RPS, RTAIL)])

        plsc.subcore_barrier()

        if with_norm:
            sc_r = [sgb_v[0, pl.ds(col * 16, 16)] for col in range(D // 16)]
            bi_r = [sgb_v[1, pl.ds(col * 16, 16)] for col in range(D // 16)]

        def gather_desc(j, b):
            return pltpu.make_async_copy(
                h_hbm.at[srcall_v.at[pl.ds(j * C, C)]], hrow_v[b],
                sems.at[0, b])

        def e_desc(j, b):
            return pltpu.make_async_copy(
                e_hbm.at[pl.ds(base + j * C, C)], e_v[b],
                sems.at[1, b])

        def dst_desc(j, b):
            return pltpu.make_async_copy(
                dst_hbm.at[pl.ds(base + j * C, C)], dst_v[b],
                sems.at[2, b])

        def scat_wait(b):
            pltpu.make_async_copy(
                hrow_v[b], aggr_sh.at[dst_v[b]], sems.at[3, b]).wait()

        def issue(j, b):
            dst_desc(j, b).start()
            e_desc(j, b).start()
            gather_desc(j, b).start()

        def consume(j, b):
            gather_desc(j, b).wait()
            e_desc(j, b).wait()

            @plsc.parallel_loop(0, C, unroll=4)
            def _(i):
                for col in range(D // 16):
                    sl = pl.ds(col * 16, 16)
                    hv = hrow_v[b][i, sl]
                    if with_norm:
                        hv = jnp.maximum(hv * sc_r[col] + bi_r[col], 0.0)
                    hrow_v[b][i, sl] = jnp.maximum(hv + e_v[b][i, sl], 0.0)

            dst_desc(j, b).wait()
            pltpu.async_copy(hrow_v[b], aggr_sh.at[dst_v[b]],
                             sems.at[3, b], add=True)

        issue(0, 0)
        issue(1, 1)

        def trip(i, carry):
            j0 = 3 * i
            for t in range(3):
                j = j0 + t
                consume(j, t)
                b2 = (t + 2) % 3

                @pl.when((j >= 1) & (j + 2 < NCHUNK))
                def _():
                    scat_wait(b2)

                @pl.when(j + 2 < NCHUNK)
                def _():
                    issue(j + 2, b2)

            return carry

        lax.fori_loop(0, NCHUNK // 3, trip, 0)
        consume(NCHUNK - 1, (NCHUNK - 1) % 3)
        scat_wait((NCHUNK - 3) % 3)
        scat_wait((NCHUNK - 2) % 3)
        scat_wait((NCHUNK - 1) % 3)
        plsc.subcore_barrier()
        pltpu.sync_copy(aggr_sh.at[pl.ds(s * RPS, RPS)],
                        out_hbm.at[c, pl.ds(s * RPS, RPS)])

        @pl.when(s == NS - 1)
        def _():
            pltpu.sync_copy(aggr_sh.at[pl.ds(NS * RPS, RTAIL)],
                            out_hbm.at[c, pl.ds(NS * RPS, RTAIL)])

    return k


_EDGE_GS_PLAIN = _make_edge_gather_scatter(False)
_EDGE_GS_NORM = _make_edge_gather_scatter(True)


BE = 4000


def _edge_lin_body(attr_ref, we_ref, be_ref, out_ref):
    out_ref[...] = (jnp.dot(attr_ref[...], we_ref[...],
                            preferred_element_type=jnp.float32)
                    + be_ref[...])


def _edge_lin(edge_attr, we, be):
    return pl.pallas_call(
        _edge_lin_body,
        grid=(E // BE,),
        in_specs=[
            pl.BlockSpec((BE, ED), lambda i: (i, 0)),
            pl.BlockSpec((ED, D), lambda i: (0, 0)),
            pl.BlockSpec((1, D), lambda i: (0, 0)),
        ],
        out_specs=pl.BlockSpec((BE, D), lambda i: (i, 0)),
        out_shape=jax.ShapeDtypeStruct((E, D), jnp.float32),
    )(edge_attr, we, be.reshape(1, D))


BN = 1000
NB = N // BN


def _make_node_mlp(with_norm: bool):
    def body(h_ref, sgb_ref, p_ref, wa_ref, ba_ref, wb_ref, bb_ref,
             g_ref, bt_ref, y_ref, so_ref):
        i = pl.program_id(0)
        hv = h_ref[...]
        if with_norm:
            hv = jnp.maximum(hv * sgb_ref[0:1, :] + sgb_ref[1:2, :], 0.0)
        hpre = hv + p_ref[0] + p_ref[1]
        t = jnp.maximum(
            jnp.dot(hpre, wa_ref[...], preferred_element_type=jnp.float32)
            + ba_ref[...], 0.0)
        y = (jnp.dot(t, wb_ref[...], preferred_element_type=jnp.float32)
             + bb_ref[...])
        y_ref[...] = y
        s1 = jnp.sum(y, axis=0, keepdims=True)
        s2 = jnp.sum(y * y, axis=0, keepdims=True)
        upd = jnp.concatenate([s1, s2, jnp.zeros((6, H), jnp.float32)],
                              axis=0)

        @pl.when(i == 0)
        def _():
            so_ref[...] = upd

        @pl.when(i > 0)
        def _():
            so_ref[...] = so_ref[...] + upd

        @pl.when(i == NB - 1)
        def _():
            mu = so_ref[0:1, :] * (1.0 / N)
            var = so_ref[1:2, :] * (1.0 / N) - mu * mu
            scale = lax.rsqrt(var + 1e-5) * g_ref[...]
            bias = bt_ref[...] - mu * scale
            so_ref[...] = jnp.concatenate(
                [scale, bias, jnp.zeros((6, H), jnp.float32)], axis=0)

    def call(h, sgb, p, wa, ba, wb, bb, g, bt):
        return pl.pallas_call(
            body,
            grid=(NB,),
            in_specs=[
                pl.BlockSpec((BN, D), lambda i: (i, 0)),
                pl.BlockSpec((8, D), lambda i: (0, 0)),
                pl.BlockSpec((NC, BN, D), lambda i: (0, i, 0)),
                pl.BlockSpec((D, H), lambda i: (0, 0)),
                pl.BlockSpec((1, H), lambda i: (0, 0)),
                pl.BlockSpec((H, H), lambda i: (0, 0)),
                pl.BlockSpec((1, H), lambda i: (0, 0)),
                pl.BlockSpec((1, H), lambda i: (0, 0)),
                pl.BlockSpec((1, H), lambda i: (0, 0)),
            ],
            out_specs=[
                pl.BlockSpec((BN, H), lambda i: (i, 0)),
                pl.BlockSpec((8, H), lambda i: (0, 0)),
            ],
            out_shape=[
                jax.ShapeDtypeStruct((N, H), jnp.float32),
                jax.ShapeDtypeStruct((8, H), jnp.float32),
            ],
        )(h, sgb, p, wa, ba.reshape(1, H), wb, bb.reshape(1, H),
          g.reshape(1, H), bt.reshape(1, H))

    return call


_NODE_MLP_PLAIN = _make_node_mlp(False)
_NODE_MLP_NORM = _make_node_mlp(True)


def _pool_body(y_ref, sgb_ref, b_ref, wo_ref, bo_ref, out_ref,
               acc_ref, cnt_ref):
    i = pl.program_id(0)
    h = jnp.maximum(y_ref[...] * sgb_ref[0:1, :] + sgb_ref[1:2, :], 0.0)
    b = b_ref[0, 0, :]
    gids = lax.broadcasted_iota(jnp.int32, (G, BN), 0)
    onehot = (b[None, :] == gids).astype(jnp.float32)
    pacc = jnp.dot(onehot, h, preferred_element_type=jnp.float32)
    pcnt = jnp.sum(onehot, axis=1, keepdims=True)

    @pl.when(i == 0)
    def _():
        acc_ref[...] = pacc
        cnt_ref[...] = pcnt

    @pl.when(i > 0)
    def _():
        acc_ref[...] = acc_ref[...] + pacc
        cnt_ref[...] = cnt_ref[...] + pcnt

    @pl.when(i == NB - 1)
    def _():
        pooled = acc_ref[...] / jnp.maximum(cnt_ref[...], 1.0)
        out_ref[...] = (jnp.dot(pooled, wo_ref[...],
                                preferred_element_type=jnp.float32)
                        + bo_ref[...])


def _pool(y, sgb, batch, wo, bo):
    b3 = batch.reshape(NB, 1, BN)
    return pl.pallas_call(
        _pool_body,
        grid=(NB,),
        in_specs=[
            pl.BlockSpec((BN, H), lambda i: (i, 0)),
            pl.BlockSpec((8, H), lambda i: (0, 0)),
            pl.BlockSpec((1, 1, BN), lambda i: (i, 0, 0)),
            pl.BlockSpec((H, 1), lambda i: (0, 0)),
            pl.BlockSpec((1, 1), lambda i: (0, 0)),
        ],
        out_specs=pl.BlockSpec((G, 1), lambda i: (0, 0)),
        out_shape=jax.ShapeDtypeStruct((G, 1), jnp.float32),
        scratch_shapes=[
            pltpu.VMEM((G, H), jnp.float32),
            pltpu.VMEM((G, 1), jnp.float32),
        ],
    )(y, sgb, b3, wo, bo.reshape(1, 1))


def kernel(x, edge_index, edge_attr, batch,
           We1, be1, W1a, b1a, W1b, b1b, gamma1, beta1,
           We2, be2, W2a, b2a, W2b, b2b, gamma2, beta2,
           We3, be3, W3a, b3a, W3b, b3b, gamma3, beta3,
           Wout, bout):
    src = edge_index[0]
    dst = edge_index[1]
    zeros = jnp.zeros((N, D), jnp.float32)
    sgb0 = jnp.zeros((8, D), jnp.float32)

    e1 = _edge_lin(edge_attr, We1, be1)
    p1 = _EDGE_GS_PLAIN(x, sgb0, e1, src, dst, zeros)
    e2 = _edge_lin(edge_attr, We2, be2)
    y1, sgb1 = _NODE_MLP_PLAIN(x, sgb0, p1, W1a, b1a, W1b, b1b,
                               gamma1, beta1)
    p2 = _EDGE_GS_NORM(y1, sgb1, e2, src, dst, zeros)
    e3 = _edge_lin(edge_attr, We3, be3)
    y2, sgb2 = _NODE_MLP_NORM(y1, sgb1, p2, W2a, b2a, W2b, b2b,
                              gamma2, beta2)
    p3 = _EDGE_GS_NORM(y2, sgb2, e3, src, dst, zeros)
    y3, sgb3 = _NODE_MLP_NORM(y2, sgb2, p3, W3a, b3a, W3b, b3b,
                              gamma3, beta3)
    return _pool(y3, sgb3, batch, Wout, bout)
